# Initial kernel scaffold; baseline (speedup 1.0000x reference)
import jax
import jax.numpy as jnp
from jax.experimental import pallas as pl


def kernel(x, edge_index, edge_attr, ew1, eb1, ew2, eb2, ew3, eb3, nw1, nb1, nw2, nb2, nw3, nb3):
    raise NotImplementedError("write your pallas kernel here")


# speedup vs baseline: 3.1703x; 3.1703x over previous
import functools

import jax
import jax.numpy as jnp
from jax import lax
from jax.experimental import pallas as pl
from jax.experimental.pallas import tpu as pltpu
from jax.experimental.pallas import tpu_sc as plsc

_NC = 2
_NS = 16
_NW = _NC * _NS
_ECHUNK = 128

_BF = jnp.bfloat16
_F32 = jnp.float32


def _mm(a, w):
    return jnp.dot(a.astype(_BF), w.astype(_BF), preferred_element_type=_F32)


def _pq_body(x_ref, wa_ref, wb_ref, p_ref, q_ref):
    xb = x_ref[...]
    p_ref[...] = _mm(xb, wa_ref[...])
    q_ref[...] = _mm(xb, wb_ref[...])


def _compute_pq(x, wa, wb):
    n, d = x.shape
    blk = 2000
    grid = n // blk
    return pl.pallas_call(
        _pq_body,
        grid=(grid,),
        in_specs=[
            pl.BlockSpec((blk, d), lambda i: (i, 0)),
            pl.BlockSpec((d, d), lambda i: (0, 0)),
            pl.BlockSpec((d, d), lambda i: (0, 0)),
        ],
        out_specs=[
            pl.BlockSpec((blk, d), lambda i: (i, 0)),
            pl.BlockSpec((blk, d), lambda i: (i, 0)),
        ],
        out_shape=[
            jax.ShapeDtypeStruct((n, d), _F32),
            jax.ShapeDtypeStruct((n, d), _F32),
        ],
    )(x, wa, wb)


def _sc_gather(p, q, src, dst):
    e = src.shape[0]
    d = p.shape[1]
    nchunks = e // _ECHUNK
    niter = (nchunks + _NW - 1) // _NW
    mesh = plsc.VectorSubcoreMesh(core_axis_name="c", subcore_axis_name="s")

    @functools.partial(
        pl.kernel,
        mesh=mesh,
        out_type=(
            jax.ShapeDtypeStruct((e, d), _F32),
            jax.ShapeDtypeStruct((e, d), _F32),
        ),
        scratch_types=[
            pltpu.VMEM((_ECHUNK,), jnp.int32),
            pltpu.VMEM((_ECHUNK,), jnp.int32),
            pltpu.VMEM((_ECHUNK, d), _F32),
            pltpu.VMEM((_ECHUNK, d), _F32),
            pltpu.SemaphoreType.DMA,
            pltpu.SemaphoreType.DMA,
        ],
    )
    def k(p_hbm, q_hbm, src_hbm, dst_hbm, ps_hbm, qd_hbm,
          isrc, idst, bufp, bufq, sem1, sem2):
        wid = lax.axis_index("s") * _NC + lax.axis_index("c")

        @pl.loop(0, niter)
        def _(j):
            ck = j * _NW + wid

            @pl.when(ck < nchunks)
            def _():
                base = ck * _ECHUNK
                pltpu.sync_copy(src_hbm.at[pl.ds(base, _ECHUNK)], isrc)
                pltpu.sync_copy(dst_hbm.at[pl.ds(base, _ECHUNK)], idst)
                cp1 = pltpu.async_copy(p_hbm.at[isrc], bufp, sem1)
                cp2 = pltpu.async_copy(q_hbm.at[idst], bufq, sem2)
                cp1.wait()
                cp2.wait()
                pltpu.sync_copy(bufp, ps_hbm.at[pl.ds(base, _ECHUNK)])
                pltpu.sync_copy(bufq, qd_hbm.at[pl.ds(base, _ECHUNK)])

    return k(p, q, src, dst)


def _edge_body(ps_ref, qd_ref, ea_ref, w1c_ref, b1_ref, w2_ref, b2_ref,
               w3_ref, b3_ref, out_ref):
    ea = ea_ref[...]
    h1 = jnp.maximum(
        ps_ref[...] + qd_ref[...] + _mm(ea, w1c_ref[...]) + b1_ref[...], 0.0)
    h2 = jnp.maximum(_mm(h1, w2_ref[...]) + b2_ref[...], 0.0)
    out_ref[...] = ea + _mm(h2, w3_ref[...]) + b3_ref[...]


def _edge_mlp(ps, qd, ea, w1c, b1, w2, b2, w3, b3):
    e, d = ea.shape
    blk = 2560
    grid = e // blk
    wspec = pl.BlockSpec((d, d), lambda i: (0, 0))
    bspec = pl.BlockSpec((1, d), lambda i: (0, 0))
    espec = pl.BlockSpec((blk, d), lambda i: (i, 0))
    return pl.pallas_call(
        _edge_body,
        grid=(grid,),
        in_specs=[espec, espec, espec, wspec, bspec, wspec, bspec, wspec,
                  bspec],
        out_specs=espec,
        out_shape=jax.ShapeDtypeStruct((e, d), _F32),
    )(ps, qd, ea, w1c, b1.reshape(1, d), w2, b2.reshape(1, d), w3,
      b3.reshape(1, d))


def _sc_scatter(ea_new, dst, n):
    e, d = ea_new.shape
    nchunks = e // _ECHUNK
    per_core = nchunks // _NC
    niter = (per_core + _NS - 1) // _NS
    wtiles = 10
    rows_per_tile = n // wtiles
    zrows = 200
    mesh = plsc.VectorSubcoreMesh(core_axis_name="c", subcore_axis_name="s")

    @functools.partial(
        pl.kernel,
        mesh=mesh,
        out_type=jax.ShapeDtypeStruct((_NC, n, d), _F32),
        scratch_types=[
            pltpu.VMEM_SHARED((n, d), _F32),
            pltpu.VMEM((zrows, d), _F32),
            pltpu.VMEM((_ECHUNK,), jnp.int32),
            pltpu.VMEM((_ECHUNK, d), _F32),
        ],
    )
    def k(ean_hbm, dst_hbm, out_hbm, acc_shared, zbuf, idxv, rowsv):
        cid = lax.axis_index("c")
        sid = lax.axis_index("s")

        @pl.loop(0, zrows)
        def _(r):
            @pl.loop(0, d // 16)
            def _(c):
                zbuf[r, pl.ds(c * 16, 16)] = jnp.zeros((16,), _F32)

        @pl.when(sid < wtiles)
        def _():
            @pl.loop(0, rows_per_tile // zrows)
            def _(j):
                pltpu.sync_copy(
                    zbuf, acc_shared.at[pl.ds(sid * rows_per_tile + j * zrows,
                                              zrows)])

        plsc.subcore_barrier()

        @pl.loop(0, niter)
        def _(j):
            lck = j * _NS + sid

            @pl.when(lck < per_core)
            def _():
                base = (cid * per_core + lck) * _ECHUNK
                pltpu.sync_copy(dst_hbm.at[pl.ds(base, _ECHUNK)], idxv)
                pltpu.sync_copy(ean_hbm.at[pl.ds(base, _ECHUNK)], rowsv)
                pltpu.sync_copy(rowsv, acc_shared.at[idxv], add=True)

        plsc.subcore_barrier()

        @pl.when(sid < wtiles)
        def _():
            pltpu.sync_copy(
                acc_shared.at[pl.ds(sid * rows_per_tile, rows_per_tile)],
                out_hbm.at[cid, pl.ds(sid * rows_per_tile, rows_per_tile)])

    return k(ea_new, dst)


def _node_body(x_ref, p0_ref, p1_ref, w1a_ref, w1b_ref, b1_ref, w2_ref,
               b2_ref, w3_ref, b3_ref, out_ref):
    xv = x_ref[...]
    agg = p0_ref[0] + p1_ref[0]
    h1 = jnp.maximum(
        _mm(xv, w1a_ref[...]) + _mm(agg, w1b_ref[...]) + b1_ref[...], 0.0)
    h2 = jnp.maximum(_mm(h1, w2_ref[...]) + b2_ref[...], 0.0)
    out_ref[...] = xv + _mm(h2, w3_ref[...]) + b3_ref[...]


def _node_mlp(x, parts, w1a, w1b, b1, w2, b2, w3, b3):
    n, d = x.shape
    blk = 2000
    grid = n // blk
    nspec = pl.BlockSpec((blk, d), lambda i: (i, 0))
    pspec = pl.BlockSpec((1, blk, d), lambda i: (0, i, 0))
    wspec = pl.BlockSpec((d, d), lambda i: (0, 0))
    bspec = pl.BlockSpec((1, d), lambda i: (0, 0))
    return pl.pallas_call(
        _node_body,
        grid=(grid,),
        in_specs=[nspec, pspec, pspec, wspec, wspec, bspec, wspec, bspec,
                  wspec, bspec],
        out_specs=nspec,
        out_shape=jax.ShapeDtypeStruct((n, d), _F32),
    )(x, parts[0:1], parts[1:2], w1a, w1b, b1.reshape(1, d), w2,
      b2.reshape(1, d), w3, b3.reshape(1, d))


def kernel(x, edge_index, edge_attr, ew1, eb1, ew2, eb2, ew3, eb3,
           nw1, nb1, nw2, nb2, nw3, nb3):
    n, d = x.shape
    src = edge_index[0]
    dst = edge_index[1]
    p, q = _compute_pq(x, ew1[:d], ew1[d:2 * d])
    ps, qd = _sc_gather(p, q, src, dst)
    ea_new = _edge_mlp(ps, qd, edge_attr, ew1[2 * d:], eb1, ew2, eb2, ew3,
                       eb3)
    parts = _sc_scatter(ea_new, dst, n)
    x_new = _node_mlp(x, parts, nw1[:d], nw1[d:], nb1, nw2, nb2, nw3, nb3)
    return x_new, ea_new
